# Initial kernel scaffold; baseline (speedup 1.0000x reference)
import jax
import jax.numpy as jnp
from jax.experimental import pallas as pl


def kernel(action, table):
    raise NotImplementedError("write your pallas kernel here")


# speedup vs baseline: 4.8091x; 4.8091x over previous
import functools

import jax
import jax.numpy as jnp
from jax import lax
from jax.experimental import pallas as pl
from jax.experimental.pallas import tpu as pltpu
from jax.experimental.pallas import tpu_sc as plsc

BATCH = 16384
HIST = 200
EMBED_DIM = 32
TOTAL = BATCH * HIST

LANES = 128
NW = 32
ROWS_TOTAL = TOTAL // LANES
ROWS_PER_W = ROWS_TOTAL // NW
K = 8
CHUNK = K * LANES
NCHUNK = ROWS_PER_W // K


def _make_sc_gather():
    mesh = plsc.VectorSubcoreMesh(core_axis_name="c", subcore_axis_name="s")

    @functools.partial(
        pl.kernel,
        out_type=jax.ShapeDtypeStruct((TOTAL, EMBED_DIM), jnp.float32),
        mesh=mesh,
        scratch_types=[
            pltpu.VMEM((K, LANES), jnp.int32),
            pltpu.VMEM((CHUNK, EMBED_DIM), jnp.float32),
            pltpu.SemaphoreType.DMA,
        ],
        compiler_params=pltpu.CompilerParams(use_tc_tiling_on_sc=False),
    )
    def gather_kernel(idx_hbm, table_hbm, out_hbm, idx_v, rows_v, sem):
        wid = lax.axis_index("s") * 2 + lax.axis_index("c")
        base_row = wid * ROWS_PER_W

        def chunk_body(c, carry):
            row0 = base_row + c * K
            pltpu.sync_copy(idx_hbm.at[pl.ds(row0, K)], idx_v)
            copies = [
                pltpu.async_copy(
                    table_hbm.at[idx_v.at[j]],
                    rows_v.at[pl.ds(j * LANES, LANES)],
                    sem,
                )
                for j in range(K)
            ]
            for cp in copies:
                cp.wait()
            pltpu.sync_copy(rows_v, out_hbm.at[pl.ds(row0 * LANES, CHUNK)])
            return carry

        lax.fori_loop(0, NCHUNK, chunk_body, 0, unroll=False)

    return gather_kernel


_sc_gather = _make_sc_gather()


@jax.jit
def kernel(action, table):
    idx = action.reshape(ROWS_TOTAL, LANES)
    out = _sc_gather(idx, table)
    return out.reshape(BATCH, HIST, EMBED_DIM)


# speedup vs baseline: 5.0475x; 1.0496x over previous
import functools

import jax
import jax.numpy as jnp
from jax import lax
from jax.experimental import pallas as pl
from jax.experimental.pallas import tpu as pltpu
from jax.experimental.pallas import tpu_sc as plsc

BATCH = 16384
HIST = 200
EMBED_DIM = 32
TOTAL = BATCH * HIST

LANES = 128
NW = 32
ROWS_TOTAL = TOTAL // LANES
ROWS_PER_W = ROWS_TOTAL // NW
K = 8
CHUNK = K * LANES
NCHUNK = ROWS_PER_W // K


def _make_sc_gather():
    mesh = plsc.VectorSubcoreMesh(core_axis_name="c", subcore_axis_name="s")

    @functools.partial(
        pl.kernel,
        out_type=jax.ShapeDtypeStruct((TOTAL, EMBED_DIM), jnp.float32),
        mesh=mesh,
        scratch_types=[
            pltpu.VMEM((2, K, LANES), jnp.int32),
            pltpu.VMEM((2, CHUNK, EMBED_DIM), jnp.float32),
            pltpu.SemaphoreType.DMA,
            pltpu.SemaphoreType.DMA,
            pltpu.SemaphoreType.DMA,
            pltpu.SemaphoreType.DMA,
            pltpu.SemaphoreType.DMA,
        ],
        compiler_params=pltpu.CompilerParams(use_tc_tiling_on_sc=False),
    )
    def gather_kernel(idx_hbm, table_hbm, out_hbm, idx_v, rows_v,
                      sem_i0, sem_i1, sem_g0, sem_g1, sem_o):
        sems_i = (sem_i0, sem_i1)
        sems_g = (sem_g0, sem_g1)
        wid = lax.axis_index("s") * 2 + lax.axis_index("c")
        base_row = wid * ROWS_PER_W

        def idx_row0(c):
            return base_row + c * K

        def load_idx(c, b):
            return pltpu.async_copy(
                idx_hbm.at[pl.ds(idx_row0(c), K)], idx_v.at[b], sems_i[b])

        def wait_idx(b):
            pltpu.make_async_copy(
                idx_hbm.at[pl.ds(base_row, K)], idx_v.at[b], sems_i[b]).wait()

        def fire_gathers(b):
            for j in range(K):
                pltpu.async_copy(
                    table_hbm.at[idx_v.at[b, j]],
                    rows_v.at[b, pl.ds(j * LANES, LANES)],
                    sems_g[b])

        def drain_gathers(b):
            for j in range(K):
                pltpu.make_async_copy(
                    table_hbm.at[idx_v.at[b, j]],
                    rows_v.at[b, pl.ds(j * LANES, LANES)],
                    sems_g[b]).wait()

        def out_slice(c):
            return out_hbm.at[pl.ds(idx_row0(c) * LANES, CHUNK)]

        def store_rows(c, b):
            pltpu.async_copy(rows_v.at[b], out_slice(c), sem_o)

        def wait_store():
            pltpu.make_async_copy(rows_v.at[0], out_slice(0), sem_o).wait()

        def step(c, b, *, first=False, fire_next=True, load_next=True):
            nb = 1 - b
            if not first:
                wait_store()
            if fire_next:
                wait_idx(nb)
                fire_gathers(nb)
            drain_gathers(b)
            store_rows(c, b)
            if load_next:
                load_idx(c + 2, b)

        load_idx(0, 0)
        wait_idx(0)
        fire_gathers(0)
        load_idx(1, 1)
        step(0, 0, first=True)

        def pair_body(p, carry):
            c = 2 * p + 1
            step(c, 1)
            step(c + 1, 0)
            return carry

        lax.fori_loop(0, (NCHUNK - 4) // 2, pair_body, 0, unroll=False)

        step(NCHUNK - 3, 1)
        step(NCHUNK - 2, 0, load_next=False)
        step(NCHUNK - 1, 1, fire_next=False, load_next=False)
        wait_store()

    return gather_kernel


_sc_gather = _make_sc_gather()


@jax.jit
def kernel(action, table):
    idx = action.reshape(ROWS_TOTAL, LANES)
    out = _sc_gather(idx, table)
    return out.reshape(BATCH, HIST, EMBED_DIM)
